# Initial kernel scaffold; baseline (speedup 1.0000x reference)
import jax
import jax.numpy as jnp
from jax.experimental import pallas as pl


def kernel(user_edge_index_0, user_edge_index_1, item_edge_index_0, item_edge_index_1, user_emb, item_emb, u_w_0, u_w_1, i_w_0, i_w_1, ua_w1, ua_b1, ua_w2, ia_w1, ia_b1, ia_w2):
    raise NotImplementedError("write your pallas kernel here")


# speedup vs baseline: 3.0771x; 3.0771x over previous
import functools
import jax
import jax.numpy as jnp
from jax import lax
from jax.experimental import pallas as pl
from jax.experimental.pallas import tpu as pltpu
from jax.experimental.pallas import tpu_sc as plsc

N = 50000
E = 800000
D = 64
HID = 128

NC, NS = 2, 16
NW = NC * NS
N_PAD = 51200
EW = 25600
E_PAD = NW * EW
CS = 1024
SCH = EW // CS
ROWS_PER_SUB = N_PAD // NS


def _sc_mesh():
  return plsc.VectorSubcoreMesh(core_axis_name="c", subcore_axis_name="s")


def _degree_kernel(edges_hbm, zeros_hbm, deg_hbm, stage_v, ones_v, deg_sh, sem):
  c = lax.axis_index("c")
  s = lax.axis_index("s")
  for r in range(128 // 16):
    ones_v[pl.ds(r * 16, 16)] = jnp.ones((16,), jnp.float32)
  rows_tile = (E_PAD // NS) // 128
  for k in range(4):
    a = 4 * c + k

    @pl.when(s == 0)
    def _():
      pltpu.sync_copy(zeros_hbm, deg_sh)

    plsc.subcore_barrier()

    def chunk_body(j, _):
      row0 = s * rows_tile + j * 16
      pltpu.sync_copy(edges_hbm.at[a, pl.ds(row0, 16), :], stage_v)
      for r in range(16):
        pltpu.sync_copy(ones_v, deg_sh.at[stage_v.at[r]], add=True)
      return 0

    lax.fori_loop(0, rows_tile // 16, chunk_body, 0)
    plsc.subcore_barrier()

    @pl.when(s == 0)
    def _():
      pltpu.sync_copy(deg_sh, deg_hbm.at[pl.ds(a * N_PAD, N_PAD)])

    plsc.subcore_barrier()


def _run_degrees(edges):
  zeros = jnp.zeros((N_PAD,), jnp.float32)
  kern = pl.kernel(
      _degree_kernel,
      out_type=jax.ShapeDtypeStruct((8 * N_PAD,), jnp.float32),
      mesh=_sc_mesh(),
      scratch_types=[
          pltpu.VMEM((16, 128), jnp.int32),
          pltpu.VMEM((128,), jnp.float32),
          pltpu.VMEM_SHARED((N_PAD,), jnp.float32),
          pltpu.SemaphoreType.DMA,
      ],
  )
  return kern(edges, zeros)


def _relayout_kernel(h128_hbm, h16_hbm, relay_v, rows_v, sem):
  c = lax.axis_index("c")
  s = lax.axis_index("s")
  wid = s * NC + c
  for q in range(4):
    for g in range(4):
      def rl_chunk(ch, _, q=q, g=g):
        r0 = wid * 200 + ch * 40
        pltpu.sync_copy(h128_hbm.at[q, g, pl.ds(r0, 40), :], relay_v)

        def wcopy(j, _):
          rows_v[j, :] = relay_v[j // 8, pl.ds((j % 8) * 16, 16)]
          return 0

        lax.fori_loop(0, 320, wcopy, 0)
        pltpu.sync_copy(rows_v.at[pl.ds(0, 320)],
                        h16_hbm.at[q, g, pl.ds(r0 * 8, 320), :])
        return 0

      lax.fori_loop(0, 5, rl_chunk, 0)


def _run_relayout(h128):
  kern = pl.kernel(
      _relayout_kernel,
      out_type=jax.ShapeDtypeStruct((4, 4, N_PAD, 16), jnp.float32),
      mesh=_sc_mesh(),
      compiler_params=pltpu.CompilerParams(use_tc_tiling_on_sc=False),
      scratch_types=[
          pltpu.VMEM((40, 128), jnp.float32),
          pltpu.VMEM((320, 16), jnp.float32),
          pltpu.SemaphoreType.DMA,
      ],
  )
  return kern(h128)


RL_SUB_ROWS = (N_PAD // 8) // NS
RL_CHUNK = 40
RL_NCH = RL_SUB_ROWS // RL_CHUNK


def _message_kernel(g, p, h128_hbm, edges_hbm, zeros_hbm, agg_hbm, h16_hbm,
                    relay_v, rl_rows_v, sidx_v, didx_v, rows_v, agg_sh, sem):
  c = lax.axis_index("c")
  s = lax.axis_index("s")
  q = 2 * c + p
  pltpu.sync_copy(zeros_hbm,
                  agg_sh.at[pl.ds(ROWS_PER_SUB * s, ROWS_PER_SUB)])

  def rl_chunk(ch, _):
    r0 = s * RL_SUB_ROWS + ch * RL_CHUNK
    pltpu.sync_copy(h128_hbm.at[q, g, pl.ds(r0, RL_CHUNK), :], relay_v)

    def wcopy(j, _):
      rl_rows_v[j, :] = relay_v[j // 8, pl.ds((j % 8) * 16, 16)]
      return 0

    lax.fori_loop(0, RL_CHUNK * 8, wcopy, 0)
    pltpu.sync_copy(rl_rows_v.at[pl.ds(0, RL_CHUNK * 8)],
                    h16_hbm.at[c, pl.ds(r0 * 8, RL_CHUNK * 8), :])
    return 0

  lax.fori_loop(0, RL_NCH, rl_chunk, 0)
  plsc.subcore_barrier()

  def chunk_body(j, _):
    row0 = s * ((E_PAD // NS) // 128) + j * 8
    pltpu.sync_copy(edges_hbm.at[2 * g, pl.ds(row0, 8), :], sidx_v)
    pltpu.sync_copy(edges_hbm.at[2 * g + 1, pl.ds(row0, 8), :], didx_v)
    descs = []
    for t in range(8):
      descs.append(pltpu.async_copy(
          h16_hbm.at[c].at[sidx_v.at[t]],
          rows_v.at[pl.ds(t * 128, 128)], sem))
    for d in descs:
      d.wait()
    for t in range(8):
      pltpu.sync_copy(rows_v.at[pl.ds(t * 128, 128)],
                      agg_sh.at[didx_v.at[t]], add=True)
    return 0

  lax.fori_loop(0, ((E_PAD // NS) // 128) // 8, chunk_body, 0)
  plsc.subcore_barrier()
  pltpu.sync_copy(agg_sh.at[pl.ds(ROWS_PER_SUB * s, ROWS_PER_SUB)],
                  agg_hbm.at[c, pl.ds(ROWS_PER_SUB * s, ROWS_PER_SUB), :])
  plsc.subcore_barrier()


def _run_messages_one(h128, edges, g, p):
  zeros = jnp.zeros((ROWS_PER_SUB, 16), jnp.float32)
  kern = pl.kernel(
      functools.partial(_message_kernel, g, p),
      out_type=[
          jax.ShapeDtypeStruct((2, N_PAD, 16), jnp.float32),
          jax.ShapeDtypeStruct((2, N_PAD, 16), jnp.float32),
      ],
      mesh=_sc_mesh(),
      compiler_params=pltpu.CompilerParams(use_tc_tiling_on_sc=False),
      scratch_types=[
          pltpu.VMEM((RL_CHUNK, 128), jnp.float32),
          pltpu.VMEM((RL_CHUNK * 8, 16), jnp.float32),
          pltpu.VMEM((8, 128), jnp.int32),
          pltpu.VMEM((8, 128), jnp.int32),
          pltpu.VMEM((CS, 16), jnp.float32),
          pltpu.VMEM_SHARED((N_PAD, 16), jnp.float32),
          pltpu.SemaphoreType.DMA,
      ],
  )
  agg, _ = kern(h128, edges, zeros)
  return agg


def _run_messages(h128, edges):
  aggs = []
  for g in range(4):
    o0 = _run_messages_one(h128, edges, g, 0)
    o1 = _run_messages_one(h128, edges, g, 1)
    aggs.append(jnp.stack([o0[0], o1[0], o0[1], o1[1]]))
  return jnp.stack(aggs)


BN = 512
NB_ALL = N_PAD // BN
NB_USED = 98


def _prescale_body(emb_ref, deg_ref, h128_ref):
  sc = lax.rsqrt(jnp.maximum(deg_ref[0, :, 0:1], 1.0))
  e = emb_ref[0]
  for qq in range(4):
    h128_ref[qq, 0] = e[:, 16 * qq:16 * (qq + 1)] * sc


def _run_prescale(embs_pad, deg4):
  grid = (4, NB_ALL)
  return pl.pallas_call(
      _prescale_body,
      grid=grid,
      in_specs=[
          pl.BlockSpec((1, BN, 64), lambda g, nb: (g // 2, nb, 0)),
          pl.BlockSpec((1, BN, 2), lambda g, nb: (g, nb, 0)),
      ],
      out_specs=pl.BlockSpec((4, 1, BN, 16), lambda g, nb: (0, g, nb, 0)),
      out_shape=jax.ShapeDtypeStruct((4, 4, N_PAD, 16), jnp.float32),
  )(embs_pad, deg4)


def _post_body(agg_ref, deg_ref, emb_ref, w_ref, aw1_ref, ab1_ref, aw2_ref,
               n_ref, s_ref):
  nb = pl.program_id(1)
  rid = lax.broadcasted_iota(jnp.int32, (BN, 1), 0) + nb * BN
  m = rid < N
  emb = emb_ref[0]
  w1 = aw1_ref[0]
  b1 = ab1_ref[0]
  w2 = aw2_ref[0]
  scs = []
  for t in range(2):
    sd = lax.rsqrt(jnp.maximum(deg_ref[t, :, 1:2], 1.0))
    w = w_ref[t]
    acc = jnp.zeros((BN, 64), jnp.float32)
    for qq in range(4):
      acc += jnp.dot(agg_ref[t, qq] * sd, w[16 * qq:16 * (qq + 1), :],
                     preferred_element_type=jnp.float32)
    o = jax.nn.relu(acc)
    nrm = jnp.sqrt(jnp.sum(o * o, axis=1, keepdims=True))
    n = o / jnp.maximum(nrm, 1e-12)
    n = jnp.where(m, n, 0.0)
    n_ref[t] = n
    h = jnp.tanh(
        jnp.dot(emb, w1[:64, :], preferred_element_type=jnp.float32)
        + jnp.dot(n, w1[64:, :], preferred_element_type=jnp.float32)
        + b1)
    hv = jnp.dot(h, w2, preferred_element_type=jnp.float32)
    scs.append(jnp.sum(jnp.where(m, hv, 0.0)))

  ri = lax.broadcasted_iota(jnp.int32, (8, 128), 0)
  li = lax.broadcasted_iota(jnp.int32, (8, 128), 1)
  upd = (jnp.where((ri == 0) & (li == 0), scs[0], 0.0)
         + jnp.where((ri == 0) & (li == 1), scs[1], 0.0))

  @pl.when(nb == 0)
  def _():
    s_ref[0] = upd

  @pl.when(nb != 0)
  def _():
    s_ref[0] += upd


def _run_post(agg, deg4, embs_pad, wstack, aw1s, ab1s, aw2s):
  grid = (2, NB_USED)
  return pl.pallas_call(
      _post_body,
      grid=grid,
      in_specs=[
          pl.BlockSpec((2, 4, BN, 16), lambda grp, nb: (grp, 0, nb, 0)),
          pl.BlockSpec((2, BN, 2), lambda grp, nb: (grp, nb, 0)),
          pl.BlockSpec((1, BN, 64), lambda grp, nb: (grp, nb, 0)),
          pl.BlockSpec((2, 64, 64), lambda grp, nb: (grp, 0, 0)),
          pl.BlockSpec((1, 128, 128), lambda grp, nb: (grp, 0, 0)),
          pl.BlockSpec((1, 1, 128), lambda grp, nb: (grp, 0, 0)),
          pl.BlockSpec((1, 128, 1), lambda grp, nb: (grp, 0, 0)),
      ],
      out_specs=[
          pl.BlockSpec((2, BN, 64), lambda grp, nb: (grp, nb, 0)),
          pl.BlockSpec((1, 8, 128), lambda grp, nb: (grp, 0, 0)),
      ],
      out_shape=[
          jax.ShapeDtypeStruct((4, N_PAD, 64), jnp.float32),
          jax.ShapeDtypeStruct((2, 8, 128), jnp.float32),
      ],
  )(agg, deg4, embs_pad, wstack, aw1s, ab1s, aw2s)


def _combine_body(grp, emb_ref, n_ref, s_ref, out_ref):
  w0 = s_ref[grp, 0, 0] / N
  w1 = s_ref[grp, 0, 1] / N
  mx = jnp.maximum(w0, w1)
  e0 = jnp.exp(w0 - mx)
  e1 = jnp.exp(w1 - mx)
  b0 = e0 / (e0 + e1)
  b1 = e1 / (e0 + e1)
  out_ref[:, :64] = emb_ref[...]
  out_ref[:, 64:] = b0 * n_ref[0] + b1 * n_ref[1]


def _run_combine(grp, emb, n_all, s_out):
  nb = pl.cdiv(N, BN)
  return pl.pallas_call(
      functools.partial(_combine_body, grp),
      grid=(nb,),
      in_specs=[
          pl.BlockSpec((BN, 64), lambda b: (b, 0)),
          pl.BlockSpec((2, BN, 64), lambda b: (grp, b, 0)),
          pl.BlockSpec((2, 8, 128), lambda b: (0, 0, 0)),
      ],
      out_specs=pl.BlockSpec((BN, 128), lambda b: (b, 0)),
      out_shape=jax.ShapeDtypeStruct((N, 128), jnp.float32),
  )(emb, n_all, s_out)


def kernel(user_edge_index_0, user_edge_index_1, item_edge_index_0,
           item_edge_index_1, user_emb, item_emb, u_w_0, u_w_1, i_w_0, i_w_1,
           ua_w1, ua_b1, ua_w2, ia_w1, ia_b1, ia_w2):
  graphs = [user_edge_index_0, user_edge_index_1,
            item_edge_index_0, item_edge_index_1]
  padn = E_PAD - E
  pad_idx = (N + jnp.arange(padn, dtype=jnp.int32) % (N_PAD - N))
  rows = []
  for ei in graphs:
    ei = ei.astype(jnp.int32)
    rows.append(jnp.concatenate([ei[0], pad_idx]))
    rows.append(jnp.concatenate([ei[1], pad_idx]))
  edges = jnp.stack(rows).reshape(8, E_PAD // 128, 128)

  embs_pad = jnp.zeros((2, N_PAD, 64), jnp.float32)
  embs_pad = embs_pad.at[0, :N].set(user_emb).at[1, :N].set(item_emb)

  deg = _run_degrees(edges)
  deg4 = jnp.transpose(deg.reshape(4, 2, N_PAD), (0, 2, 1))

  h4 = _run_prescale(embs_pad, deg4)
  h128 = jnp.reshape(h4, (4, 4, N_PAD // 8, 128))
  agg = _run_messages(h128, edges)

  wstack = jnp.stack([u_w_0, u_w_1, i_w_0, i_w_1])
  aw1s = jnp.stack([ua_w1, ia_w1])
  ab1s = jnp.stack([ua_b1, ia_b1]).reshape(2, 1, 128)
  aw2s = jnp.stack([ua_w2, ia_w2])
  n_all, s_out = _run_post(agg, deg4, embs_pad, wstack, aw1s, ab1s, aw2s)

  user_out = _run_combine(0, user_emb, n_all, s_out)
  item_out = _run_combine(1, item_emb, n_all, s_out)
  return (user_out, item_out)
